# Initial kernel scaffold; baseline (speedup 1.0000x reference)
import jax
import jax.numpy as jnp
from jax.experimental import pallas as pl


def kernel(x, ei, Wl, bl, Wr, gamma, beta):
    raise NotImplementedError("write your pallas kernel here")


# speedup vs baseline: 3.9014x; 3.9014x over previous
import functools

import jax
import jax.numpy as jnp
from jax import lax
from jax.experimental import pallas as pl
from jax.experimental.pallas import tpu as pltpu
from jax.experimental.pallas import tpu_sc as plsc

N_NODES = 10000
D = 256
DH = 128
N_EDGES = 160000
EPS = 1e-5

NT = 16
EPT = N_EDGES // NT
CHUNK = 128
NCHUNK = 80
EPT_PAD = NCHUNK * CHUNK
N_ACC = 10240
ROWS_PER_TILE = N_ACC // NT


def _sc_segment_sum(src_t, dst_t, x_lo, x_hi, zeros128, ones128):
    mesh = plsc.VectorSubcoreMesh(core_axis_name="c", subcore_axis_name="s")

    @functools.partial(
        pl.kernel,
        mesh=mesh,
        out_type=(
            jax.ShapeDtypeStruct((N_ACC, DH), jnp.float32),
            jax.ShapeDtypeStruct((N_ACC, DH), jnp.float32),
            jax.ShapeDtypeStruct((N_ACC,), jnp.float32),
        ),
        scratch_types=[
            pltpu.VMEM((NCHUNK, CHUNK), jnp.int32),
            pltpu.VMEM((NCHUNK, CHUNK), jnp.int32),
            pltpu.VMEM((CHUNK, DH), jnp.float32),
            pltpu.VMEM((CHUNK,), jnp.float32),
            pltpu.VMEM_SHARED((N_ACC, DH), jnp.float32),
            pltpu.VMEM_SHARED((N_ACC,), jnp.float32),
            pltpu.SemaphoreType.DMA,
        ],
    )
    def seg_sum(src_hbm, dst_hbm, xlo_hbm, xhi_hbm, z_hbm, o_hbm,
                sumlo_hbm, sumhi_hbm, cnt_hbm,
                src_v, dst_v, rows_v, ones_v, acc_sh, cnt_sh, sem):
        c = lax.axis_index("c")
        s = lax.axis_index("s")

        pltpu.sync_copy(z_hbm, rows_v)
        pltpu.sync_copy(o_hbm, ones_v)
        pltpu.sync_copy(src_hbm.at[s], src_v)
        pltpu.sync_copy(dst_hbm.at[s], dst_v)

        base = s * ROWS_PER_TILE
        for k in range(ROWS_PER_TILE // CHUNK):
            pltpu.sync_copy(rows_v, acc_sh.at[pl.ds(base + k * CHUNK, CHUNK)])
            pltpu.sync_copy(rows_v.at[0], cnt_sh.at[pl.ds(base + k * CHUNK, CHUNK)])
        plsc.subcore_barrier()

        @pl.when(c == 0)
        def _():
            def step(j, carry):
                pltpu.async_copy(xlo_hbm.at[src_v.at[j]], rows_v, sem).wait()
                pltpu.sync_copy(rows_v, acc_sh.at[dst_v.at[j]], add=True)
                pltpu.sync_copy(ones_v, cnt_sh.at[dst_v.at[j]], add=True)
                return carry
            lax.fori_loop(0, NCHUNK, step, 0)

        @pl.when(c == 1)
        def _():
            def step(j, carry):
                pltpu.async_copy(xhi_hbm.at[src_v.at[j]], rows_v, sem).wait()
                pltpu.sync_copy(rows_v, acc_sh.at[dst_v.at[j]], add=True)
                return carry
            lax.fori_loop(0, NCHUNK, step, 0)

        plsc.subcore_barrier()

        rsl = pl.ds(base, ROWS_PER_TILE)

        @pl.when(c == 0)
        def _():
            pltpu.sync_copy(acc_sh.at[rsl], sumlo_hbm.at[rsl])
            pltpu.sync_copy(cnt_sh.at[rsl], cnt_hbm.at[rsl])

        @pl.when(c == 1)
        def _():
            pltpu.sync_copy(acc_sh.at[rsl], sumhi_hbm.at[rsl])

    return seg_sum(src_t, dst_t, x_lo, x_hi, zeros128, ones128)


BR = 1000
NB = N_NODES // BR


def _tc_conv_body(sumlo_ref, sumhi_ref, cnt_ref, x_ref, wl_ref, bl_ref, wr_ref,
                  conv_ref, stats_ref, acc_ref):
    i = pl.program_id(0)
    recip = 1.0 / jnp.maximum(cnt_ref[...], 1.0)
    mlo = sumlo_ref[...] * recip
    mhi = sumhi_ref[...] * recip
    wl = wl_ref[...]
    dn = (((1,), (1,)), ((), ()))
    conv = lax.dot_general(mlo, wl[:, :DH], dn, preferred_element_type=jnp.float32)
    conv += lax.dot_general(mhi, wl[:, DH:], dn, preferred_element_type=jnp.float32)
    conv += lax.dot_general(x_ref[...], wr_ref[...], dn,
                            preferred_element_type=jnp.float32)
    conv += bl_ref[...]
    conv_ref[...] = conv

    @pl.when(i == 0)
    def _():
        acc_ref[...] = jnp.zeros_like(acc_ref)

    acc_ref[0:1, :] += jnp.sum(conv, axis=0, keepdims=True)
    acc_ref[1:2, :] += jnp.sum(conv * conv, axis=0, keepdims=True)
    stats_ref[...] = acc_ref[...]


def _tc_bn_body(conv_ref, stats_ref, x_ref, gamma_ref, beta_ref, out_ref):
    n = jnp.float32(N_NODES)
    s1 = stats_ref[0:1, :]
    s2 = stats_ref[1:2, :]
    mu = s1 / n
    var = s2 / n - mu * mu
    inv = lax.rsqrt(var + EPS)
    bn = gamma_ref[...] * (conv_ref[...] - mu) * inv + beta_ref[...]
    out_ref[...] = jnp.maximum(bn, 0.0) + x_ref[...]


def kernel(x, ei, Wl, bl, Wr, gamma, beta):
    src = ei[0].astype(jnp.int32)
    dst = ei[1].astype(jnp.int32)

    pad = EPT_PAD - EPT
    src_t = jnp.concatenate(
        [src.reshape(NT, EPT), jnp.zeros((NT, pad), jnp.int32)], axis=1
    ).reshape(NT, NCHUNK, CHUNK)
    dst_t = jnp.concatenate(
        [dst.reshape(NT, EPT), jnp.full((NT, pad), N_NODES, jnp.int32)], axis=1
    ).reshape(NT, NCHUNK, CHUNK)

    x_lo = x[:, :DH]
    x_hi = x[:, DH:]
    zeros128 = jnp.zeros((CHUNK, DH), jnp.float32)
    ones128 = jnp.ones((CHUNK,), jnp.float32)

    sum_lo, sum_hi, cnt = _sc_segment_sum(src_t, dst_t, x_lo, x_hi, zeros128, ones128)
    cnt2 = cnt.reshape(N_ACC, 1)

    conv, stats = pl.pallas_call(
        _tc_conv_body,
        grid=(NB,),
        in_specs=[
            pl.BlockSpec((BR, DH), lambda i: (i, 0)),
            pl.BlockSpec((BR, DH), lambda i: (i, 0)),
            pl.BlockSpec((BR, 1), lambda i: (i, 0)),
            pl.BlockSpec((BR, D), lambda i: (i, 0)),
            pl.BlockSpec((D, D), lambda i: (0, 0)),
            pl.BlockSpec((1, D), lambda i: (0, 0)),
            pl.BlockSpec((D, D), lambda i: (0, 0)),
        ],
        out_specs=[
            pl.BlockSpec((BR, D), lambda i: (i, 0)),
            pl.BlockSpec((8, D), lambda i: (0, 0)),
        ],
        out_shape=[
            jax.ShapeDtypeStruct((N_NODES, D), jnp.float32),
            jax.ShapeDtypeStruct((8, D), jnp.float32),
        ],
        scratch_shapes=[pltpu.VMEM((8, D), jnp.float32)],
    )(sum_lo, sum_hi, cnt2, x, Wl, bl.reshape(1, D), Wr)

    out = pl.pallas_call(
        _tc_bn_body,
        grid=(NB,),
        in_specs=[
            pl.BlockSpec((BR, D), lambda i: (i, 0)),
            pl.BlockSpec((8, D), lambda i: (0, 0)),
            pl.BlockSpec((BR, D), lambda i: (i, 0)),
            pl.BlockSpec((1, D), lambda i: (0, 0)),
            pl.BlockSpec((1, D), lambda i: (0, 0)),
        ],
        out_specs=pl.BlockSpec((BR, D), lambda i: (i, 0)),
        out_shape=jax.ShapeDtypeStruct((N_NODES, D), jnp.float32),
    )(conv, stats, x, gamma.reshape(1, D), beta.reshape(1, D))

    return out
